# baseline (speedup 1.0000x reference)
import jax
import jax.numpy as jnp
from jax import lax
from jax.experimental import pallas as pl
from jax.experimental.pallas import tpu as pltpu
from jax.experimental.pallas import tpu_sc as plsc

VOCAB = 1000000
DIM = 64
BATCH = 16384

NUM_CORES = 2
NUM_SUBCORES = 16
LANES = 16
NUM_WORKERS = NUM_CORES * NUM_SUBCORES
B_PER_W = BATCH // NUM_WORKERS
NBLK = (VOCAB + 127) // 128
BLK_PER_W = 256
SEG = 1024
RING = 8
STAGE_ROWS = BATCH + 128
ROWBUF = 128


def _worker_id():
    return lax.axis_index("s") * NUM_CORES + lax.axis_index("c")


def _id_at(ref, i):
    return ref[pl.ds(i, LANES)][0]


def _gather_table(w, ids_hbm, table_hbm, stage_hbm,
                  ids_all, sel_pos, blk_ring, rows_v, pos_v,
                  cnt_s, off_s, ord_s, bsems, ssem, lane, chunk_c8, chunk_cm):
    pltpu.sync_copy(ids_hbm, ids_all.at[pl.ds(0, BATCH)])
    dump = jnp.int32(BATCH) + w
    dump_v = jnp.broadcast_to(dump, (LANES,))

    def scan_chunk(c, off):
        v = ids_all[pl.ds(c * LANES, LANES)]
        own = ((v >> 7) >> 8) == w
        pos = c * LANES + lane
        plsc.store_compressed(sel_pos.at[pl.ds(off, LANES)], pos, mask=own)
        cnt = plsc.all_reduce_population_count(own)[0]
        return off + cnt

    nsel = lax.fori_loop(0, BATCH // LANES, scan_chunk, jnp.int32(0))

    nseg = (nsel + (SEG - 1)) // SEG

    def segment(seg, _):
        k0 = seg * SEG
        klen = jnp.minimum(jnp.int32(SEG), nsel - k0)

        def zero(b, _):
            cnt_s[b] = jnp.int32(0)
            return 0

        lax.fori_loop(0, BLK_PER_W + 1, zero, 0)

        nfull = klen // LANES

        def count16(c, _):
            pv = sel_pos[pl.ds(k0 + c * LANES, LANES)]
            blv = (plsc.load_gather(ids_all, [pv]) >> 7) - w * BLK_PER_W
            for j in range(LANES):
                bl = blv[j]
                cnt_s[bl + 1] = cnt_s[bl + 1] + 1
            return 0

        lax.fori_loop(0, nfull, count16, 0)

        def count(k, _):
            pos = _id_at(sel_pos, k0 + k)
            bl = (_id_at(ids_all, pos) >> 7) - w * BLK_PER_W
            cnt_s[bl + 1] = cnt_s[bl + 1] + 1
            return 0

        lax.fori_loop(nfull * LANES, klen, count, 0)

        def prefix(b, _):
            cnt_s[b + 1] = cnt_s[b + 1] + cnt_s[b]
            off_s[b] = cnt_s[b]
            return 0

        lax.fori_loop(0, BLK_PER_W, prefix, 0)

        def place16(c, _):
            pv = sel_pos[pl.ds(k0 + c * LANES, LANES)]
            blv = (plsc.load_gather(ids_all, [pv]) >> 7) - w * BLK_PER_W
            for j in range(LANES):
                bl = blv[j]
                slot = off_s[bl]
                off_s[bl] = slot + 1
                ord_s[slot] = pv[j]
            return 0

        lax.fori_loop(0, nfull, place16, 0)

        def place(k, _):
            pos = _id_at(sel_pos, k0 + k)
            bl = (_id_at(ids_all, pos) >> 7) - w * BLK_PER_W
            slot = off_s[bl]
            off_s[bl] = slot + 1
            ord_s[slot] = pos
            return 0

        lax.fori_loop(nfull * LANES, klen, place, 0)

        def issue_blk(b, u):
            bc = jnp.minimum(jnp.int32(b), jnp.int32(BLK_PER_W - 1))

            @pl.when(jnp.logical_and(b < BLK_PER_W,
                                     cnt_s[bc + 1] > cnt_s[bc]))
            def _():
                rb = (w * BLK_PER_W + bc) * 128
                pltpu.async_copy(
                    table_hbm.at[:, :, pl.ds(pl.multiple_of(rb, 128), 128)],
                    blk_ring.at[u], bsems[u])

        for u in range(RING):
            issue_blk(jnp.int32(u), u)

        def reset_posv():
            for q in range(ROWBUF // LANES):
                pos_v[pl.ds(q * LANES, LANES)] = dump_v

        reset_posv()

        def flush(j, posacc, force):
            jn = jnp.where(j == ROWBUF, 0, j)

            @pl.when(jnp.logical_or(j == ROWBUF, jnp.logical_and(
                force, j > 0)))
            def _():
                @pl.when(j % LANES != 0)
                def _():
                    pos_v[pl.ds((j // LANES) * LANES, LANES)] = posacc
                pltpu.sync_copy(rows_v, stage_hbm.at[pos_v])
                reset_posv()

            pacc = jnp.where(jnp.logical_or(j == ROWBUF, force),
                             dump_v, posacc)
            return jn, pacc

        def bgroup(g, carry):
            for u in range(RING):
                b = g * RING + u

                def process(carry):
                    j, posacc = carry
                    pltpu.make_async_copy(
                        table_hbm.at[:, :, pl.ds(0, 128)],
                        blk_ring.at[u], bsems[u]).wait()

                    def per_id(k, c2):
                        j, posacc = c2
                        pos = ord_s[k]
                        idk = _id_at(ids_all, pos)
                        rrv = jnp.broadcast_to(idk & 127, (LANES,))
                        for q in range(4):
                            vals = plsc.load_gather(
                                blk_ring.at[u],
                                [chunk_c8[q], chunk_cm[q], rrv])
                            rows_v[j, pl.ds(q * LANES, LANES)] = vals
                        posacc = jnp.where(lane == (j % LANES),
                                           jnp.broadcast_to(pos, (LANES,)),
                                           posacc)

                        @pl.when(j % LANES == LANES - 1)
                        def _():
                            pos_v[pl.ds((j // LANES) * LANES, LANES)] = posacc

                        posacc = jnp.where(j % LANES == LANES - 1,
                                           dump_v, posacc)
                        j, posacc = flush(j + 1, posacc, jnp.bool_(False))
                        return j, posacc

                    return lax.fori_loop(
                        cnt_s[b], cnt_s[b + 1], per_id, (j, posacc))

                hit = cnt_s[b + 1] > cnt_s[b]
                carry = lax.cond(hit, process, lambda c: c, carry)
                issue_blk(b + RING, u)
            return carry

        carry = (jnp.int32(0), dump_v)
        carry = lax.fori_loop(0, BLK_PER_W // RING, bgroup, carry)
        flush(carry[0], carry[1], jnp.bool_(True))
        return 0

    lax.fori_loop(0, nseg, segment, 0)


def _phase_a(center_hbm, context_hbm, embt_hbm, ctxt_hbm,
             stage_c_hbm, stage_x_hbm,
             ids_all, sel_pos, blk_ring, rows_v, pos_v,
             cnt_s, off_s, ord_s, bsems, ssem):
    w = _worker_id()
    lane = lax.iota(jnp.int32, LANES)
    chunk_c8 = [(jnp.int32(16 * q) + lane) >> 3 for q in range(4)]
    chunk_cm = [(jnp.int32(16 * q) + lane) & 7 for q in range(4)]
    _gather_table(w, center_hbm, embt_hbm, stage_c_hbm,
                  ids_all, sel_pos, blk_ring, rows_v, pos_v,
                  cnt_s, off_s, ord_s, bsems, ssem, lane, chunk_c8, chunk_cm)
    _gather_table(w, context_hbm, ctxt_hbm, stage_x_hbm,
                  ids_all, sel_pos, blk_ring, rows_v, pos_v,
                  cnt_s, off_s, ord_s, bsems, ssem, lane, chunk_c8, chunk_cm)


def _phase_b(stage_c_hbm, stage_x_hbm, out_hbm, cen_b, ctx_b, out_v,
             csems, xsems):
    w = _worker_id()
    base = w * B_PER_W
    lane = lax.iota(jnp.int32, LANES)
    col_c = [jnp.broadcast_to(jnp.int32(c), (LANES,)) for c in range(DIM)]
    nchunk = B_PER_W // 128

    def issue(chunk):
        u = chunk % 2
        r0 = base + chunk * 128
        pltpu.async_copy(stage_c_hbm.at[pl.ds(r0, 128), :],
                         cen_b.at[u], csems[u])
        pltpu.async_copy(stage_x_hbm.at[pl.ds(r0, 128), :],
                         ctx_b.at[u], xsems[u])

    issue(0)
    for chunk in range(nchunk):
        u = chunk % 2
        pltpu.make_async_copy(stage_c_hbm.at[pl.ds(0, 128), :],
                              cen_b.at[u], csems[u]).wait()
        pltpu.make_async_copy(stage_x_hbm.at[pl.ds(0, 128), :],
                              ctx_b.at[u], xsems[u]).wait()
        if chunk + 1 < nchunk:
            issue(chunk + 1)

        def grp(g, _):
            rows = g * LANES + lane
            part = [jnp.zeros((LANES,), jnp.float32) for _ in range(4)]
            for c in range(DIM):
                a = plsc.load_gather(cen_b.at[u], [rows, col_c[c]])
                bb = plsc.load_gather(ctx_b.at[u], [rows, col_c[c]])
                part[c % 4] = part[c % 4] + a * bb
            out_v[pl.ds(chunk * 128 + g * LANES, LANES)] = (
                (part[0] + part[1]) + (part[2] + part[3]))
            return 0

        lax.fori_loop(0, 128 // LANES, grp, 0)

    pltpu.sync_copy(out_v, out_hbm.at[pl.ds(base, B_PER_W)])


@jax.jit
def kernel(center_ids, context_ids, embeddings, context_embeddings):
    mesh = plsc.VectorSubcoreMesh(
        core_axis_name="c", subcore_axis_name="s",
        num_cores=NUM_CORES, num_subcores=NUM_SUBCORES)
    params = pltpu.CompilerParams(needs_layout_passes=False)

    gather = pl.kernel(
        _phase_a,
        out_type=(jax.ShapeDtypeStruct((STAGE_ROWS, 128), jnp.float32),
                  jax.ShapeDtypeStruct((STAGE_ROWS, 128), jnp.float32)),
        mesh=mesh,
        scratch_types=[
            pltpu.VMEM((BATCH + LANES,), jnp.int32),
            pltpu.VMEM((BATCH + LANES,), jnp.int32),
            pltpu.VMEM((RING, 8, 8, 128), jnp.float32),
            pltpu.VMEM((ROWBUF, 128), jnp.float32),
            pltpu.VMEM((ROWBUF,), jnp.int32),
            pltpu.SMEM((BLK_PER_W + 1,), jnp.int32),
            pltpu.SMEM((BLK_PER_W,), jnp.int32),
            pltpu.SMEM((SEG,), jnp.int32),
            [pltpu.SemaphoreType.DMA] * RING,
            pltpu.SemaphoreType.DMA,
        ],
        compiler_params=params,
    )
    dot = pl.kernel(
        _phase_b,
        out_type=jax.ShapeDtypeStruct((BATCH,), jnp.float32),
        mesh=mesh,
        scratch_types=[
            pltpu.VMEM((2, 128, 128), jnp.float32),
            pltpu.VMEM((2, 128, 128), jnp.float32),
            pltpu.VMEM((B_PER_W,), jnp.float32),
            [pltpu.SemaphoreType.DMA] * 2,
            [pltpu.SemaphoreType.DMA] * 2,
        ],
        compiler_params=params,
    )

    embt = embeddings.T.reshape(8, 8, VOCAB)
    ctxt = context_embeddings.T.reshape(8, 8, VOCAB)
    stage_c, stage_x = gather(center_ids.astype(jnp.int32),
                              context_ids.astype(jnp.int32), embt, ctxt)
    return dot(stage_c, stage_x)


# speedup vs baseline: 1.0107x; 1.0107x over previous
import jax
import jax.numpy as jnp
from jax import lax
from jax.experimental import pallas as pl
from jax.experimental.pallas import tpu as pltpu
from jax.experimental.pallas import tpu_sc as plsc

VOCAB = 1000000
DIM = 64
BATCH = 16384

NUM_CORES = 2
NUM_SUBCORES = 16
LANES = 16
NUM_WORKERS = NUM_CORES * NUM_SUBCORES
B_PER_W = BATCH // NUM_WORKERS
NBLK = (VOCAB + 127) // 128
BLK_PER_W = 256
SEG = 1024
RING = 8
STAGE_ROWS = BATCH + 128
ROWBUF = 128


def _worker_id():
    return lax.axis_index("s") * NUM_CORES + lax.axis_index("c")


def _id_at(ref, i):
    return ref[pl.ds(i, LANES)][0]


def _gather_table(w, ids_hbm, table_hbm, stage_hbm,
                  ids_all, sel_pos, blk_ring, rows_v, pos_v,
                  cnt_s, off_s, ord_s, bsems, ssem, lane, chunk_c8, chunk_cm):
    pltpu.sync_copy(ids_hbm, ids_all.at[pl.ds(0, BATCH)])
    dump = jnp.int32(BATCH) + w
    dump_v = jnp.broadcast_to(dump, (LANES,))

    def scan_chunk(c4, off):
        for s in range(4):
            c = c4 * 4 + s
            v = ids_all[pl.ds(c * LANES, LANES)]
            own = ((v >> 7) >> 8) == w
            pos = c * LANES + lane
            plsc.store_compressed(sel_pos.at[pl.ds(off, LANES)], pos,
                                  mask=own)
            off = off + plsc.all_reduce_population_count(own)[0]
        return off

    nsel = lax.fori_loop(0, BATCH // LANES // 4, scan_chunk, jnp.int32(0))

    nseg = (nsel + (SEG - 1)) // SEG

    def segment(seg, _):
        k0 = seg * SEG
        klen = jnp.minimum(jnp.int32(SEG), nsel - k0)

        def zero(b, _):
            cnt_s[b] = jnp.int32(0)
            return 0

        lax.fori_loop(0, BLK_PER_W + 1, zero, 0)

        nfull = klen // LANES

        def count16(c, _):
            pv = sel_pos[pl.ds(k0 + c * LANES, LANES)]
            blv = (plsc.load_gather(ids_all, [pv]) >> 7) - w * BLK_PER_W
            for j in range(LANES):
                bl = blv[j]
                cnt_s[bl + 1] = cnt_s[bl + 1] + 1
            return 0

        lax.fori_loop(0, nfull, count16, 0)

        def count(k, _):
            pos = _id_at(sel_pos, k0 + k)
            bl = (_id_at(ids_all, pos) >> 7) - w * BLK_PER_W
            cnt_s[bl + 1] = cnt_s[bl + 1] + 1
            return 0

        lax.fori_loop(nfull * LANES, klen, count, 0)

        def prefix(b, _):
            cnt_s[b + 1] = cnt_s[b + 1] + cnt_s[b]
            off_s[b] = cnt_s[b]
            return 0

        lax.fori_loop(0, BLK_PER_W, prefix, 0)

        def place16(c, _):
            pv = sel_pos[pl.ds(k0 + c * LANES, LANES)]
            blv = (plsc.load_gather(ids_all, [pv]) >> 7) - w * BLK_PER_W
            for j in range(LANES):
                bl = blv[j]
                slot = off_s[bl]
                off_s[bl] = slot + 1
                ord_s[slot] = pv[j]
            return 0

        lax.fori_loop(0, nfull, place16, 0)

        def place(k, _):
            pos = _id_at(sel_pos, k0 + k)
            bl = (_id_at(ids_all, pos) >> 7) - w * BLK_PER_W
            slot = off_s[bl]
            off_s[bl] = slot + 1
            ord_s[slot] = pos
            return 0

        lax.fori_loop(nfull * LANES, klen, place, 0)

        def issue_blk(b, u):
            bc = jnp.minimum(jnp.int32(b), jnp.int32(BLK_PER_W - 1))

            @pl.when(jnp.logical_and(b < BLK_PER_W,
                                     cnt_s[bc + 1] > cnt_s[bc]))
            def _():
                rb = (w * BLK_PER_W + bc) * 128
                pltpu.async_copy(
                    table_hbm.at[:, :, pl.ds(pl.multiple_of(rb, 128), 128)],
                    blk_ring.at[u], bsems[u])

        for u in range(RING):
            issue_blk(jnp.int32(u), u)

        def reset_posv():
            for q in range(ROWBUF // LANES):
                pos_v[pl.ds(q * LANES, LANES)] = dump_v

        reset_posv()

        def flush(j, posacc, force):
            jn = jnp.where(j == ROWBUF, 0, j)

            @pl.when(jnp.logical_or(j == ROWBUF, jnp.logical_and(
                force, j > 0)))
            def _():
                @pl.when(j % LANES != 0)
                def _():
                    pos_v[pl.ds((j // LANES) * LANES, LANES)] = posacc
                pltpu.sync_copy(rows_v, stage_hbm.at[pos_v])
                reset_posv()

            pacc = jnp.where(jnp.logical_or(j == ROWBUF, force),
                             dump_v, posacc)
            return jn, pacc

        def bgroup(g, carry):
            for u in range(RING):
                b = g * RING + u

                def process(carry):
                    j, posacc = carry
                    pltpu.make_async_copy(
                        table_hbm.at[:, :, pl.ds(0, 128)],
                        blk_ring.at[u], bsems[u]).wait()

                    def per_id(k, c2):
                        j, posacc = c2
                        pos = ord_s[k]
                        idk = _id_at(ids_all, pos)
                        rrv = jnp.broadcast_to(idk & 127, (LANES,))
                        for q in range(4):
                            vals = plsc.load_gather(
                                blk_ring.at[u],
                                [chunk_c8[q], chunk_cm[q], rrv])
                            rows_v[j, pl.ds(q * LANES, LANES)] = vals
                        posacc = jnp.where(lane == (j % LANES),
                                           jnp.broadcast_to(pos, (LANES,)),
                                           posacc)

                        @pl.when(j % LANES == LANES - 1)
                        def _():
                            pos_v[pl.ds((j // LANES) * LANES, LANES)] = posacc

                        posacc = jnp.where(j % LANES == LANES - 1,
                                           dump_v, posacc)
                        j, posacc = flush(j + 1, posacc, jnp.bool_(False))
                        return j, posacc

                    return lax.fori_loop(
                        cnt_s[b], cnt_s[b + 1], per_id, (j, posacc))

                hit = cnt_s[b + 1] > cnt_s[b]
                carry = lax.cond(hit, process, lambda c: c, carry)
                issue_blk(b + RING, u)
            return carry

        carry = (jnp.int32(0), dump_v)
        carry = lax.fori_loop(0, BLK_PER_W // RING, bgroup, carry)
        flush(carry[0], carry[1], jnp.bool_(True))
        return 0

    lax.fori_loop(0, nseg, segment, 0)


def _phase_a(center_hbm, context_hbm, embt_hbm, ctxt_hbm,
             stage_c_hbm, stage_x_hbm,
             ids_all, sel_pos, blk_ring, rows_v, pos_v,
             cnt_s, off_s, ord_s, bsems, ssem):
    w = _worker_id()
    lane = lax.iota(jnp.int32, LANES)
    chunk_c8 = [(jnp.int32(16 * q) + lane) >> 3 for q in range(4)]
    chunk_cm = [(jnp.int32(16 * q) + lane) & 7 for q in range(4)]
    _gather_table(w, center_hbm, embt_hbm, stage_c_hbm,
                  ids_all, sel_pos, blk_ring, rows_v, pos_v,
                  cnt_s, off_s, ord_s, bsems, ssem, lane, chunk_c8, chunk_cm)
    _gather_table(w, context_hbm, ctxt_hbm, stage_x_hbm,
                  ids_all, sel_pos, blk_ring, rows_v, pos_v,
                  cnt_s, off_s, ord_s, bsems, ssem, lane, chunk_c8, chunk_cm)


def _phase_b(stage_c_hbm, stage_x_hbm, out_hbm, cen_b, ctx_b, out_v,
             csems, xsems):
    w = _worker_id()
    base = w * B_PER_W
    lane = lax.iota(jnp.int32, LANES)
    col_c = [jnp.broadcast_to(jnp.int32(c), (LANES,)) for c in range(DIM)]
    nchunk = B_PER_W // 128

    def issue(chunk):
        u = chunk % 2
        r0 = base + chunk * 128
        pltpu.async_copy(stage_c_hbm.at[pl.ds(r0, 128), :],
                         cen_b.at[u], csems[u])
        pltpu.async_copy(stage_x_hbm.at[pl.ds(r0, 128), :],
                         ctx_b.at[u], xsems[u])

    issue(0)
    for chunk in range(nchunk):
        u = chunk % 2
        pltpu.make_async_copy(stage_c_hbm.at[pl.ds(0, 128), :],
                              cen_b.at[u], csems[u]).wait()
        pltpu.make_async_copy(stage_x_hbm.at[pl.ds(0, 128), :],
                              ctx_b.at[u], xsems[u]).wait()
        if chunk + 1 < nchunk:
            issue(chunk + 1)

        def grp(g, _):
            rows = g * LANES + lane
            part = [jnp.zeros((LANES,), jnp.float32) for _ in range(4)]
            for c in range(DIM):
                a = plsc.load_gather(cen_b.at[u], [rows, col_c[c]])
                bb = plsc.load_gather(ctx_b.at[u], [rows, col_c[c]])
                part[c % 4] = part[c % 4] + a * bb
            out_v[pl.ds(chunk * 128 + g * LANES, LANES)] = (
                (part[0] + part[1]) + (part[2] + part[3]))
            return 0

        lax.fori_loop(0, 128 // LANES, grp, 0)

    pltpu.sync_copy(out_v, out_hbm.at[pl.ds(base, B_PER_W)])


@jax.jit
def kernel(center_ids, context_ids, embeddings, context_embeddings):
    mesh = plsc.VectorSubcoreMesh(
        core_axis_name="c", subcore_axis_name="s",
        num_cores=NUM_CORES, num_subcores=NUM_SUBCORES)
    params = pltpu.CompilerParams(needs_layout_passes=False)

    gather = pl.kernel(
        _phase_a,
        out_type=(jax.ShapeDtypeStruct((STAGE_ROWS, 128), jnp.float32),
                  jax.ShapeDtypeStruct((STAGE_ROWS, 128), jnp.float32)),
        mesh=mesh,
        scratch_types=[
            pltpu.VMEM((BATCH + LANES,), jnp.int32),
            pltpu.VMEM((BATCH + LANES,), jnp.int32),
            pltpu.VMEM((RING, 8, 8, 128), jnp.float32),
            pltpu.VMEM((ROWBUF, 128), jnp.float32),
            pltpu.VMEM((ROWBUF,), jnp.int32),
            pltpu.SMEM((BLK_PER_W + 1,), jnp.int32),
            pltpu.SMEM((BLK_PER_W,), jnp.int32),
            pltpu.SMEM((SEG,), jnp.int32),
            [pltpu.SemaphoreType.DMA] * RING,
            pltpu.SemaphoreType.DMA,
        ],
        compiler_params=params,
    )
    dot = pl.kernel(
        _phase_b,
        out_type=jax.ShapeDtypeStruct((BATCH,), jnp.float32),
        mesh=mesh,
        scratch_types=[
            pltpu.VMEM((2, 128, 128), jnp.float32),
            pltpu.VMEM((2, 128, 128), jnp.float32),
            pltpu.VMEM((B_PER_W,), jnp.float32),
            [pltpu.SemaphoreType.DMA] * 2,
            [pltpu.SemaphoreType.DMA] * 2,
        ],
        compiler_params=params,
    )

    embt = embeddings.T.reshape(8, 8, VOCAB)
    ctxt = context_embeddings.T.reshape(8, 8, VOCAB)
    stage_c, stage_x = gather(center_ids.astype(jnp.int32),
                              context_ids.astype(jnp.int32), embt, ctxt)
    return dot(stage_c, stage_x)
